# baseline (speedup 1.0000x reference)
import functools

import jax
import jax.numpy as jnp
from jax.experimental import pallas as pl
from jax.experimental.pallas import tpu as pltpu


def _rotl(x, d):
    return (x << jnp.uint32(d)) | (x >> jnp.uint32(32 - d))


def _threefry_bits(j):
    ks0 = jnp.uint32(0)
    ks1 = jnp.uint32(1)
    ks2 = jnp.uint32(0x1BD11BDA) ^ ks0 ^ ks1
    ks = (ks0, ks1, ks2)
    rotations = ((13, 15, 26, 6), (17, 29, 16, 24))
    x0 = jnp.zeros_like(j) + ks0
    x1 = j + ks1
    for i in range(5):
        for r in rotations[i % 2]:
            x0 = x0 + x1
            x1 = _rotl(x1, r)
            x1 = x1 ^ x0
        x0 = x0 + ks[(i + 1) % 3]
        x1 = x1 + ks[(i + 2) % 3] + jnp.uint32(i + 1)
    return x0 ^ x1


def _row_body(temps_ref, logits_ref, probs_ref, tok_ref, *, V, C):
    r = pl.program_id(0)
    t_raw = temps_ref[r]
    t = jnp.where(t_raw < 1e-5, jnp.float32(1.0), t_raw)

    x = logits_ref[...]
    scaled = x / t
    sm = jnp.max(scaled)
    e = jnp.exp(scaled - sm)
    s = jnp.sum(e)
    probs = e / s
    probs_ref[...] = probs

    sub = jax.lax.broadcasted_iota(jnp.int32, x.shape, 1)
    lane = jax.lax.broadcasted_iota(jnp.int32, x.shape, 2)
    flat = sub * C + lane
    j = (r * V + flat).astype(jnp.uint32)
    bits = _threefry_bits(j)

    uf = jax.lax.bitcast_convert_type(
        (bits >> jnp.uint32(9)) | jnp.uint32(0x3F800000), jnp.float32
    ) - jnp.float32(1.0)
    q = -jnp.log1p(-uf)

    ratio = probs / q
    big = jnp.int32(V)
    isnan = ratio != ratio
    nan_idx = jnp.min(jnp.where(isnan, flat, big))
    mxf = jnp.max(jnp.where(isnan, -jnp.inf, ratio))
    max_idx = jnp.min(jnp.where(ratio == mxf, flat, big))
    sampled = jnp.where(nan_idx < big, nan_idx, max_idx)

    gm = jnp.max(x)
    greedy = jnp.min(jnp.where(x == gm, flat, big))

    tok_ref[...] = jnp.reshape(jnp.where(t_raw < 1e-5, greedy, sampled), (1, 1, 1))


def kernel(logits, temperatures):
    B, V = logits.shape
    SUB = 8
    C = V // SUB
    x3 = logits.reshape(B, SUB, C)
    probs3, tok3 = pl.pallas_call(
        functools.partial(_row_body, V=V, C=C),
        grid=(B,),
        in_specs=[
            pl.BlockSpec(memory_space=pltpu.SMEM),
            pl.BlockSpec((1, SUB, C), lambda r: (r, 0, 0)),
        ],
        out_specs=[
            pl.BlockSpec((1, SUB, C), lambda r: (r, 0, 0)),
            pl.BlockSpec((1, 1, 1), lambda r: (r, 0, 0)),
        ],
        out_shape=[
            jax.ShapeDtypeStruct((B, SUB, C), jnp.float32),
            jax.ShapeDtypeStruct((B, 1, 1), jnp.int32),
        ],
    )(temperatures, x3)
    return tok3.reshape(B), probs3.reshape(B, V)


# speedup vs baseline: 1.0147x; 1.0147x over previous
import functools

import jax
import jax.numpy as jnp
from jax.experimental import pallas as pl
from jax.experimental.pallas import tpu as pltpu


def _rotl(x, d):
    return (x << jnp.uint32(d)) | (x >> jnp.uint32(32 - d))


def _threefry_bits(j):
    ks0 = jnp.uint32(0)
    ks1 = jnp.uint32(1)
    ks2 = jnp.uint32(0x1BD11BDA) ^ ks0 ^ ks1
    ks = (ks0, ks1, ks2)
    rotations = ((13, 15, 26, 6), (17, 29, 16, 24))
    x0 = jnp.zeros_like(j) + ks0
    x1 = j + ks1
    for i in range(5):
        for r in rotations[i % 2]:
            x0 = x0 + x1
            x1 = _rotl(x1, r)
            x1 = x1 ^ x0
        x0 = x0 + ks[(i + 1) % 3]
        x1 = x1 + ks[(i + 2) % 3] + jnp.uint32(i + 1)
    return x0 ^ x1


def _row_body(temps_ref, logits_ref, probs_ref, tok_ref, *, V, C):
    r = pl.program_id(0)
    t_raw = temps_ref[r]
    t = jnp.where(t_raw < 1e-5, jnp.float32(1.0), t_raw)

    x = logits_ref[...]
    scaled = x * (jnp.float32(1.0) / t)
    sm = jnp.max(scaled)
    e = jnp.exp(scaled - sm)
    s = jnp.sum(e)
    probs = e * (jnp.float32(1.0) / s)
    probs_ref[...] = probs

    sub = jax.lax.broadcasted_iota(jnp.int32, x.shape, 1)
    lane = jax.lax.broadcasted_iota(jnp.int32, x.shape, 2)
    flat = sub * C + lane
    j = (r * V + flat).astype(jnp.uint32)
    bits = _threefry_bits(j)

    uf = jax.lax.bitcast_convert_type(
        (bits >> jnp.uint32(9)) | jnp.uint32(0x3F800000), jnp.float32
    ) - jnp.float32(1.0)
    q = -jnp.log1p(-uf)

    ratio = probs / q
    big = jnp.int32(V)
    isnan = ratio != ratio
    nan_idx = jnp.min(jnp.where(isnan, flat, big))
    mxf = jnp.max(jnp.where(isnan, -jnp.inf, ratio))
    max_idx = jnp.min(jnp.where(ratio == mxf, flat, big))
    sampled = jnp.where(nan_idx < big, nan_idx, max_idx)

    greedy = jnp.min(jnp.where(scaled == sm, flat, big))

    tok_ref[...] = jnp.reshape(jnp.where(t_raw < 1e-5, greedy, sampled), (1, 1, 1))


def kernel(logits, temperatures):
    B, V = logits.shape
    SUB = 8
    C = V // SUB
    x3 = logits.reshape(B, SUB, C)
    probs3, tok3 = pl.pallas_call(
        functools.partial(_row_body, V=V, C=C),
        grid=(B,),
        in_specs=[
            pl.BlockSpec(memory_space=pltpu.SMEM),
            pl.BlockSpec((1, SUB, C), lambda r: (r, 0, 0)),
        ],
        out_specs=[
            pl.BlockSpec((1, SUB, C), lambda r: (r, 0, 0)),
            pl.BlockSpec((1, 1, 1), lambda r: (r, 0, 0)),
        ],
        out_shape=[
            jax.ShapeDtypeStruct((B, SUB, C), jnp.float32),
            jax.ShapeDtypeStruct((B, 1, 1), jnp.int32),
        ],
    )(temperatures, x3)
    return tok3.reshape(B), probs3.reshape(B, V)
